# baseline (speedup 1.0000x reference)
import functools

import jax
import jax.numpy as jnp
from jax import lax
from jax.experimental import pallas as pl
from jax.experimental.pallas import tpu as pltpu
from jax.experimental.pallas import tpu_sc as plsc

_BATCH = 4096
_HIST = 50
_D = 64
_V = 1000000
_NC = 2
_NS = 16
_NW = _NC * _NS
_RPW = _BATCH // _NW
_NBUF = 3
_CB = _D * 128

_mesh = plsc.VectorSubcoreMesh(core_axis_name="c", subcore_axis_name="s")


@functools.partial(
    pl.kernel,
    mesh=_mesh,
    out_type=jax.ShapeDtypeStruct((_HIST, 8, _NW, 1024), jnp.float32),
    compiler_params=pltpu.CompilerParams(
        use_tc_tiling_on_sc=False, needs_layout_passes=False
    ),
    scratch_types=[
        pltpu.VMEM((_HIST, _RPW), jnp.int32),
        pltpu.VMEM((_NBUF, _D, 128), jnp.int32),
        pltpu.VMEM((_NBUF, _CB), jnp.float32),
    ] + [pltpu.SemaphoreType.DMA] * (2 * _NBUF),
)
def _gather(idx_hbm, tbl_hbm, out_hbm, idx_v, iexp_v, dst_v, *sems):
    gsems = sems[:_NBUF]
    wsems = sems[_NBUF:]
    wid = lax.axis_index("s") * _NC + lax.axis_index("c")
    pltpu.sync_copy(idx_hbm.at[:, pl.ds(wid * _RPW, _RPW)], idx_v)

    def build_indices(h, nb):
        toks = tuple(idx_v[h, pl.ds(16 * i, 16)] for i in range(8))

        def d_body(d, carry):
            for i in range(8):
                iexp_v[nb, d, pl.ds(16 * i, 16)] = carry[i]
            return tuple(c + _V for c in carry)

        lax.fori_loop(0, _D, d_body, toks)

    def issue_streams(nb):
        for d in range(_D):
            pltpu.async_copy(
                tbl_hbm.at[iexp_v.at[nb, d]],
                dst_v.at[nb, pl.ds(d * 128, 128)],
                gsems[nb],
            )

    def wait_streams(nb):
        pltpu.make_async_copy(
            tbl_hbm.at[pl.ds(0, _CB)], dst_v.at[nb], gsems[nb]
        ).wait()

    def write_out(h, nb):
        for kd in range(8):
            pltpu.async_copy(
                dst_v.at[nb, pl.ds(kd * 1024, 1024)],
                out_hbm.at[h, kd, wid],
                wsems[nb],
            )

    def drain_writes(nb):
        for kd in range(8):
            pltpu.make_async_copy(
                dst_v.at[nb, pl.ds(kd * 1024, 1024)],
                out_hbm.at[0, kd, 0],
                wsems[nb],
            ).wait()

    for nb in range(_NBUF):
        build_indices(nb, nb)
        issue_streams(nb)

    _NGRP = _HIST // _NBUF

    def grp(g, carry):
        for nb in range(_NBUF):
            chunk = g * _NBUF + nb
            wait_streams(nb)
            write_out(chunk, nb)

            @pl.when(chunk + _NBUF < _HIST)
            def _():
                drain_writes(nb)
                build_indices(chunk + _NBUF, nb)
                issue_streams(nb)
        return carry

    lax.fori_loop(0, _NGRP, grp, 0)
    for k in range(_HIST - _NBUF * _NGRP):
        wait_streams(k)
        write_out(_NBUF * _NGRP + k, k)
    for nb in range(_NBUF):
        drain_writes(nb)


def kernel(word_batch, table):
    wbt = word_batch.astype(jnp.int32).T
    tbl_flat = table.T.reshape(_D * _V)
    out4 = _gather(wbt, tbl_flat)
    out5 = out4.reshape(_HIST, 8, _NW, 8, 128)
    return out5.transpose(2, 4, 0, 1, 3).reshape(_BATCH, _HIST, _D)


# speedup vs baseline: 7.0830x; 7.0830x over previous
import functools

import jax
import jax.numpy as jnp
from jax import lax
from jax.experimental import pallas as pl
from jax.experimental.pallas import tpu as pltpu
from jax.experimental.pallas import tpu_sc as plsc

_BATCH = 4096
_HIST = 50
_D = 64
_NC = 2
_NS = 16
_NW = _NC * _NS
_RPW = _BATCH // _NW
_NB = 8

_mesh = plsc.VectorSubcoreMesh(core_axis_name="c", subcore_axis_name="s")


@functools.partial(
    pl.kernel,
    mesh=_mesh,
    out_type=jax.ShapeDtypeStruct((_BATCH, _HIST, _D), jnp.float32),
    compiler_params=pltpu.CompilerParams(use_tc_tiling_on_sc=False),
    scratch_types=[
        pltpu.VMEM((_RPW, _HIST), jnp.int32),
        pltpu.VMEM((_NB * _HIST, _D), jnp.float32),
    ] + [pltpu.SemaphoreType.DMA] * _NB,
)
def _gather(idx_hbm, table_hbm, out_hbm, idx_v, rows_v, *sems):
    wid = lax.axis_index("s") * _NC + lax.axis_index("c")
    row0 = wid * _RPW
    pltpu.sync_copy(idx_hbm.at[pl.ds(row0, _RPW)], idx_v)

    def buf(b):
        return rows_v.at[pl.ds(b * _HIST, _HIST)]

    for b in range(_NB):
        pltpu.async_copy(table_hbm.at[idx_v.at[b]], buf(b), sems[b])

    def grp(g, carry):
        for b in range(_NB):
            chunk = g * _NB + b
            pltpu.make_async_copy(table_hbm.at[idx_v.at[b]], buf(b), sems[b]).wait()
            pltpu.sync_copy(buf(b), out_hbm.at[row0 + chunk])
            pltpu.async_copy(table_hbm.at[idx_v.at[chunk + _NB]], buf(b), sems[b])
        return carry

    lax.fori_loop(0, _RPW // _NB - 1, grp, 0)

    for b in range(_NB):
        chunk = _RPW - _NB + b
        pltpu.make_async_copy(table_hbm.at[idx_v.at[b]], buf(b), sems[b]).wait()
        pltpu.sync_copy(buf(b), out_hbm.at[row0 + chunk])


def kernel(word_batch, table):
    return _gather(word_batch.astype(jnp.int32), table)
